# baseline (speedup 1.0000x reference)
import functools
import math

import jax
import jax.numpy as jnp
from jax import lax
from jax.experimental import pallas as pl
from jax.experimental.pallas import tpu as pltpu
from jax.experimental.pallas import tpu_sc as plsc

VOCAB = 1000000
EMB = 64
SEQ = 2048
BATCH = 4

_NC = 2
_NS = 16
_NW = _NC * _NS
_N_TOK = SEQ * BATCH
_NBLK = VOCAB // 128
_BPW = 248
_CBLK = 4
_CLANE = _CBLK * 128
_CSHIFT = 7 + _CBLK.bit_length() - 1
_TAIL0 = _NBLK * 128
_NTAIL = VOCAB - _TAIL0
_OUT_ROWS = _N_TOK + 8
_SCALE = math.sqrt(EMB)
_I32 = jnp.int32


def _emb_kernel(tok_hbm, wt3_hbm, tail_hbm, out_hbm,
                tok_v, tokf_v, chunk_v, rows_v, pos_v, tail_v,
                sem_a, sem_b, sem_o):
    wid = lax.axis_index("s") * _NC + lax.axis_index("c")
    iota = lax.broadcasted_iota(_I32, (16,), 0)

    def splat(x):
        return jax.lax.broadcast_in_dim(jnp.asarray(x, _I32), (16,), ())

    lo = wid * _BPW
    hi = jnp.minimum(lo + _BPW, _NBLK)
    nb = hi - lo
    is31 = splat(wid == _NW - 1)

    pltpu.sync_copy(tail_hbm, tail_v)

    for b in range(2):
        for g in range(8):
            pos_v[b, pl.ds(g * 16, 16)] = jnp.full((16,), _N_TOK, _I32)

    sems = (sem_a, sem_b)
    nc = nb // _CBLK

    def chunk_copy(c, par):
        start = pl.multiple_of((lo + c * _CBLK) * 128, 128)
        return pltpu.make_async_copy(
            wt3_hbm.at[:, :, pl.ds(start, _CLANE)],
            chunk_v.at[par],
            sems[par],
        )

    chunk_copy(0, 0).start()
    chunk_copy(1, 1).start()

    lo_s, hi_s = splat(lo), splat(hi)

    lo128_s = splat(lo * 128)

    nf = jnp.asarray(0, _I32)
    for h in range(2):
        pltpu.sync_copy(tok_hbm.at[pl.ds(h * (_N_TOK // 2), _N_TOK // 2)],
                        tok_v)

        def filt(i, nf, _h=h):
            t = tok_v[pl.ds(i * 16, 16)]
            blk = t >> 7
            m = (blk >= lo_s) & (blk < hi_s)
            m = m | ((t >= _TAIL0) & (is31 > 0))
            pos = _h * (_N_TOK // 2) + i * 16 + iota
            packed = (pos << 15) | (t - lo128_s)
            plsc.store_compressed(tokf_v.at[pl.ds(nf, 16)], packed, mask=m)
            return nf + jnp.max(plsc.all_reduce_population_count(m))

        nf = lax.fori_loop(0, _N_TOK // 32, filt, nf)
    nf_s = splat(nf)
    nj = (nf + 15) >> 4

    def flush_batch(cur):
        src = rows_v.at[pl.ds(pl.multiple_of(cur * 128, 128), 128)]
        pltpu.async_copy(src, out_hbm.at[pos_v.at[cur]], sem_o).wait()
        for g in range(8):
            plsc.store_scatter(
                pos_v,
                [splat(cur), g * 16 + iota],
                jnp.full((16,), _N_TOK, _I32),
                mask=iota >= 0,
            )

    def process_hits(m0, tf, slot, load_fn):
        def maybe_flush(slot):
            def do_flush(s):
                cur = s >> 7
                flush_batch(cur)
                return ((cur + 1) & 1) << 7

            return lax.cond((slot & 127) > 111, do_flush, lambda s: s, slot)

        slot = maybe_flush(slot)
        k = jnp.max(plsc.all_reduce_population_count(m0))

        def body(_, c):
            m, slot = c
            ffs = plsc.all_reduce_ffs(m)
            e_h = jnp.take_along_axis(tf, ffs, axis=0)
            rc_h = e_h & 0x7FFF
            p_h = e_h >> 15
            for g in range(EMB // 16):
                v = load_fn(rc_h, g)
                rows_v[slot, pl.ds(g * 16, 16)] = v * _SCALE
            plsc.store_scatter(
                pos_v,
                [splat(slot >> 7), splat(slot & 127)],
                p_h,
                mask=iota < 1,
            )
            return m & (iota != ffs), slot + 1

        _, slot = lax.fori_loop(0, k, body, (m0, slot))
        return slot

    def outer(c2, slot):
        for par in (0, 1):
            c = c2 * 2 + par

            @pl.when(c < nc)
            def _():
                chunk_copy(c, par).wait()

            cid_s = splat(c)
            c_spl = splat(par)

            def load_main(rc_h, g):
                cc = g * 16 + iota
                return plsc.load_gather(
                    chunk_v, [c_spl, cc >> 3, cc & 7, rc_h & (_CLANE - 1)]
                )

            def match(j, slot):
                tf = tokf_v[pl.ds(j * 16, 16)]
                valid = (j * 16 + iota) < nf_s
                m0 = (((tf & 0x7FFF) >> _CSHIFT) == cid_s) & valid
                return process_hits(m0, tf, slot, load_main)

            slot = lax.fori_loop(0, nj, match, slot)

            @pl.when(c + 2 < nc)
            def _():
                chunk_copy(c + 2, par).start()
        return slot

    slot = lax.fori_loop(0, (_BPW // _CBLK + 1) // 2, outer,
                         jnp.asarray(0, _I32))

    tl_s = splat(_TAIL0) - lo128_s

    def load_tail(rc_h, g):
        return plsc.load_gather(tail_v, [rc_h - tl_s, g * 16 + iota])

    def match_tail(j, slot):
        tf = tokf_v[pl.ds(j * 16, 16)]
        valid = (j * 16 + iota) < nf_s
        m0 = ((tf & 0x7FFF) >= tl_s) & valid & (is31 > 0)
        return process_hits(m0, tf, slot, load_tail)

    slot = lax.fori_loop(0, nj, match_tail, slot)

    fb = slot >> 7
    src = rows_v.at[pl.ds(pl.multiple_of(fb * 128, 128), 128)]
    pltpu.async_copy(src, out_hbm.at[pos_v.at[fb]], sem_o).wait()


@jax.jit
def kernel(tokens, W):
    tok = tokens.reshape(_N_TOK).astype(_I32)
    wt3 = W.T.reshape(8, 8, VOCAB)
    tail = W[_TAIL0:, :]
    grid_kernel = pl.kernel(
        _emb_kernel,
        out_type=jax.ShapeDtypeStruct((_OUT_ROWS, 128), jnp.float32),
        mesh=plsc.VectorSubcoreMesh(core_axis_name="c", subcore_axis_name="s"),
        scratch_types=[
            pltpu.VMEM((_N_TOK // 2,), _I32),
            pltpu.VMEM((_N_TOK + 16,), _I32),
            pltpu.VMEM((2, 8, 8, _CLANE), jnp.float32),
            pltpu.VMEM((256, 128), jnp.float32),
            pltpu.VMEM((2, 128), _I32),
            pltpu.VMEM((_NTAIL, EMB), jnp.float32),
            pltpu.SemaphoreType.DMA,
            pltpu.SemaphoreType.DMA,
            pltpu.SemaphoreType.DMA,
        ],
        compiler_params=pltpu.CompilerParams(needs_layout_passes=False),
    )
    out = grid_kernel(tok, wt3, tail)
    return out[:_N_TOK, :EMB].reshape(SEQ, BATCH, EMB)


# speedup vs baseline: 1.4243x; 1.4243x over previous
import functools
import math

import jax
import jax.numpy as jnp
from jax import lax
from jax.experimental import pallas as pl
from jax.experimental.pallas import tpu as pltpu
from jax.experimental.pallas import tpu_sc as plsc

VOCAB = 1000000
EMB = 64
SEQ = 2048
BATCH = 4

_NC = 2
_NS = 16
_NW = _NC * _NS
_N_TOK = SEQ * BATCH
_NBLK = VOCAB // 128
_BPW = 248
_CBLK = 2
_CLANE = _CBLK * 128
_CSHIFT = 7 + _CBLK.bit_length() - 1
_TAIL0 = _NBLK * 128
_NTAIL = VOCAB - _TAIL0
_OUT_ROWS = _N_TOK + 8
_SCALE = math.sqrt(EMB)
_I32 = jnp.int32


def _emb_kernel(tok_hbm, wt3_hbm, tail_hbm, out_hbm,
                tok_v, tokf_v, chunk_v, rows_v, pos_v, tail_v,
                sem_a, sem_b, sem_c, sem_o):
    wid = lax.axis_index("s") * _NC + lax.axis_index("c")
    iota = lax.broadcasted_iota(_I32, (16,), 0)

    def splat(x):
        return jax.lax.broadcast_in_dim(jnp.asarray(x, _I32), (16,), ())

    lo = wid * _BPW
    hi = jnp.minimum(lo + _BPW, _NBLK)
    nb = hi - lo
    is31 = splat(wid == _NW - 1)

    pltpu.sync_copy(tok_hbm, tok_v)
    pltpu.sync_copy(tail_hbm, tail_v)

    for b in range(2):
        for g in range(8):
            pos_v[b, pl.ds(g * 16, 16)] = jnp.full((16,), _N_TOK, _I32)

    sems = (sem_a, sem_b, sem_c)
    nc = nb // _CBLK

    def chunk_copy(c, par):
        start = pl.multiple_of((lo + c * _CBLK) * 128, 128)
        return pltpu.make_async_copy(
            wt3_hbm.at[:, :, pl.ds(start, _CLANE)], chunk_v.at[par], sems[par]
        )

    chunk_copy(0, 0).start()
    chunk_copy(1, 1).start()

    lo_s, hi_s = splat(lo), splat(hi)

    lo128_s = splat(lo * 128)

    def filt(i, nf):
        t = tok_v[pl.ds(i * 16, 16)]
        blk = t >> 7
        m = (blk >= lo_s) & (blk < hi_s)
        m = m | ((t >= _TAIL0) & (is31 > 0))
        packed = ((i * 16 + iota) << 15) | (t - lo128_s)
        plsc.store_compressed(tokf_v.at[pl.ds(nf, 16)], packed, mask=m)
        return nf + jnp.max(plsc.all_reduce_population_count(m))

    nf = lax.fori_loop(0, _N_TOK // 16, filt, jnp.asarray(0, _I32))
    nf_s = splat(nf)
    nj = (nf + 15) >> 4

    def flush(slot):
        fb = slot >> 7

        def do_flush(_):
            src = rows_v.at[pl.ds(pl.multiple_of(fb * 128, 128), 128)]
            pltpu.async_copy(src, out_hbm.at[pos_v.at[fb]], sem_o).wait()
            for g in range(8):
                plsc.store_scatter(
                    pos_v,
                    [splat(fb), g * 16 + iota],
                    jnp.full((16,), _N_TOK, _I32),
                    mask=iota >= 0,
                )
            return 0

        lax.cond((slot & 127) == 127, do_flush, lambda _: 0, 0)
        return (slot + 1) & 255

    def process_hits(m0, tf, slot, load_fn):
        def cond(c):
            m, _ = c
            return jnp.any(m)

        def body(c):
            m, slot = c
            ffs = plsc.all_reduce_ffs(m)
            e_h = jnp.take_along_axis(tf, ffs, axis=0)
            rc_h = e_h & 0x7FFF
            p_h = e_h >> 15
            for g in range(EMB // 16):
                v = load_fn(rc_h, g)
                rows_v[slot, pl.ds(g * 16, 16)] = v * _SCALE
            plsc.store_scatter(
                pos_v,
                [splat(slot >> 7), splat(slot & 127)],
                p_h,
                mask=iota < 1,
            )
            return m & (iota != ffs), flush(slot)

        _, slot = lax.while_loop(cond, body, (m0, slot))
        return slot

    def outer(c3, slot):
        for par in (0, 1, 2):
            c = c3 * 3 + par

            @pl.when(c < nc)
            def _():
                chunk_copy(c, par).wait()

            @pl.when(c + 2 < nc)
            def _():
                chunk_copy(c + 2, (par + 2) % 3).start()

            cid_s = splat(c)
            c_spl = splat(par)

            def load_main(rc_h, g):
                cc = g * 16 + iota
                return plsc.load_gather(
                    chunk_v, [c_spl, cc >> 3, cc & 7, rc_h & (_CLANE - 1)]
                )

            def match(j, slot):
                tf = tokf_v[pl.ds(j * 16, 16)]
                valid = (j * 16 + iota) < nf_s
                m0 = (((tf & 0x7FFF) >> _CSHIFT) == cid_s) & valid
                return process_hits(m0, tf, slot, load_main)

            slot = lax.fori_loop(0, nj, match, slot)
        return slot

    slot = lax.fori_loop(0, (_BPW // _CBLK + 2) // 3, outer,
                         jnp.asarray(0, _I32))

    tl_s = splat(_TAIL0) - lo128_s

    def load_tail(rc_h, g):
        return plsc.load_gather(tail_v, [rc_h - tl_s, g * 16 + iota])

    def match_tail(j, slot):
        tf = tokf_v[pl.ds(j * 16, 16)]
        valid = (j * 16 + iota) < nf_s
        m0 = ((tf & 0x7FFF) >= tl_s) & valid & (is31 > 0)
        return process_hits(m0, tf, slot, load_tail)

    slot = lax.fori_loop(0, nj, match_tail, slot)

    fb = slot >> 7
    src = rows_v.at[pl.ds(pl.multiple_of(fb * 128, 128), 128)]
    pltpu.async_copy(src, out_hbm.at[pos_v.at[fb]], sem_o).wait()


@jax.jit
def kernel(tokens, W):
    tok = tokens.reshape(_N_TOK).astype(_I32)
    wt3 = W.T.reshape(8, 8, VOCAB)
    tail = W[_TAIL0:, :]
    grid_kernel = pl.kernel(
        _emb_kernel,
        out_type=jax.ShapeDtypeStruct((_OUT_ROWS, 128), jnp.float32),
        mesh=plsc.VectorSubcoreMesh(core_axis_name="c", subcore_axis_name="s"),
        scratch_types=[
            pltpu.VMEM((_N_TOK,), _I32),
            pltpu.VMEM((_N_TOK + 16,), _I32),
            pltpu.VMEM((3, 8, 8, _CLANE), jnp.float32),
            pltpu.VMEM((256, 128), jnp.float32),
            pltpu.VMEM((2, 128), _I32),
            pltpu.VMEM((_NTAIL, EMB), jnp.float32),
            pltpu.SemaphoreType.DMA,
            pltpu.SemaphoreType.DMA,
            pltpu.SemaphoreType.DMA,
            pltpu.SemaphoreType.DMA,
        ],
        compiler_params=pltpu.CompilerParams(needs_layout_passes=False),
    )
    out = grid_kernel(tok, wt3, tail)
    return out[:_N_TOK, :EMB].reshape(SEQ, BATCH, EMB)
